# baseline (speedup 1.0000x reference)
import jax
import jax.numpy as jnp
from jax.experimental import pallas as pl
from jax.experimental.pallas import tpu as pltpu

_K = 1024
_D = 256
_NTOK = 8192
_CC = 0.25
_NB = 2048


def _vq_step(x_hbm, w_ref, q_ref, enc_ref, idx_ref, loss_ref, ppl_ref,
             counts_ref, sse_ref, wnorm_ref, xbuf, xsem):
    g = pl.program_id(0)
    nsteps = pl.num_programs(0)
    w = w_ref[...]

    def _xcopy(i):
        return pltpu.make_async_copy(
            x_hbm.at[pl.ds(i * _NB, _NB), :], xbuf.at[i % 2], xsem.at[i % 2])

    @pl.when(g == 0)
    def _init():
        _xcopy(0).start()
        counts_ref[...] = jnp.zeros_like(counts_ref)
        sse_ref[0] = 0.0
        wnorm_ref[...] = jnp.sum(w * w, axis=1, keepdims=True)

    @pl.when(g + 1 < nsteps)
    def _prefetch():
        _xcopy(g + 1).start()

    _xcopy(g).wait()
    x = xbuf[g % 2]
    wnorm = wnorm_ref[...]
    xnorm = jnp.sum(x * x, axis=1, keepdims=True)
    s_t = jax.lax.dot_general(w, x, (((1,), (1,)), ((), ())),
                              preferred_element_type=jnp.float32,
                              precision=jax.lax.Precision.DEFAULT)
    dist_t = (jnp.transpose(xnorm) + wnorm) - 2.0 * s_t
    idx = jnp.argmin(dist_t, axis=0)

    iota_k = jax.lax.broadcasted_iota(jnp.int32, (_NB, _K), 1)
    idx_col = jnp.transpose(idx[None, :])
    enc = (idx_col == iota_k).astype(jnp.float32)
    q = jax.lax.dot_general(enc, w, (((1,), (0,)), ((), ())),
                            preferred_element_type=jnp.float32,
                            precision=jax.lax.Precision.DEFAULT)
    q_ref[...] = x + (q - x)
    enc_ref[...] = enc
    idx_ref[0, 0] = idx

    counts_ref[...] += jnp.sum(enc, axis=0, keepdims=True)
    sse_ref[0] += jnp.sum((q - x) ** 2)

    @pl.when(g == nsteps - 1)
    def _fini():
        loss_ref[0, 0] = _CC * sse_ref[0] / (_NTOK * _D)
        p = counts_ref[...] / _NTOK
        ent = jnp.sum(p * jnp.log(p + 1e-10))
        ppl_ref[0, 0] = jnp.exp(-ent)


def kernel(inputs, embedding_weight):
    x = jnp.transpose(inputs, (0, 2, 3, 4, 1)).reshape(_NTOK, _D)
    nsteps = _NTOK // _NB
    q, enc, idx, loss, ppl = pl.pallas_call(
        _vq_step,
        grid=(nsteps,),
        in_specs=[
            pl.BlockSpec(memory_space=pl.ANY),
            pl.BlockSpec((_K, _D), lambda g: (0, 0)),
        ],
        out_specs=[
            pl.BlockSpec((_NB, _D), lambda g: (g, 0)),
            pl.BlockSpec((_NB, _K), lambda g: (g, 0)),
            pl.BlockSpec((1, 1, _NB), lambda g: (g, 0, 0)),
            pl.BlockSpec(memory_space=pltpu.SMEM),
            pl.BlockSpec(memory_space=pltpu.SMEM),
        ],
        out_shape=[
            jax.ShapeDtypeStruct((_NTOK, _D), jnp.float32),
            jax.ShapeDtypeStruct((_NTOK, _K), jnp.float32),
            jax.ShapeDtypeStruct((nsteps, 1, _NB), jnp.int32),
            jax.ShapeDtypeStruct((1, 1), jnp.float32),
            jax.ShapeDtypeStruct((1, 1), jnp.float32),
        ],
        scratch_shapes=[
            pltpu.VMEM((1, _K), jnp.float32),
            pltpu.SMEM((1,), jnp.float32),
            pltpu.VMEM((_K, 1), jnp.float32),
            pltpu.VMEM((2, _NB, _D), jnp.float32),
            pltpu.SemaphoreType.DMA((2,)),
        ],
    )(x, embedding_weight)
    q_out = q.reshape(4, 8, 16, 16, _D).transpose(0, 4, 1, 2, 3)
    return (loss[0, 0], q_out, ppl[0, 0], enc, idx.reshape(_NTOK, 1))


# speedup vs baseline: 1.0450x; 1.0450x over previous
import jax
import jax.numpy as jnp
from jax.experimental import pallas as pl
from jax.experimental.pallas import tpu as pltpu

_K = 1024
_D = 256
_NTOK = 8192
_CC = 0.25
_NB = 2048


def _vq_step(x_ref, w_ref, q_ref, enc_ref, idx_ref, loss_ref, ppl_ref,
             counts_ref, sse_ref, wnorm_ref):
    g = pl.program_id(0)
    nsteps = pl.num_programs(0)
    w = w_ref[...]

    @pl.when(g == 0)
    def _init():
        counts_ref[...] = jnp.zeros_like(counts_ref)
        sse_ref[0] = 0.0
        wnorm_ref[...] = jnp.sum(w * w, axis=1, keepdims=True)

    x = x_ref[...]
    wnorm = wnorm_ref[...]
    xnorm = jnp.sum(x * x, axis=1, keepdims=True)
    s_t = jax.lax.dot_general(w, x, (((1,), (1,)), ((), ())),
                              preferred_element_type=jnp.float32,
                              precision=jax.lax.Precision.DEFAULT)
    dist_t = (jnp.transpose(xnorm) + wnorm) - 2.0 * s_t
    idx = jnp.argmin(dist_t, axis=0)

    iota_k = jax.lax.broadcasted_iota(jnp.int32, (_NB, _K), 1)
    idx_col = jnp.transpose(idx[None, :])
    enc = (idx_col == iota_k).astype(jnp.float32)
    q = jax.lax.dot_general(enc, w, (((1,), (0,)), ((), ())),
                            preferred_element_type=jnp.float32,
                            precision=jax.lax.Precision.DEFAULT)
    q_ref[...] = x + (q - x)
    enc_ref[...] = enc
    idx_ref[0, 0] = idx

    counts_ref[...] += jnp.sum(enc, axis=0, keepdims=True)
    sse_ref[0] += jnp.sum((q - x) ** 2)

    @pl.when(g == nsteps - 1)
    def _fini():
        loss_ref[0, 0] = _CC * sse_ref[0] / (_NTOK * _D)
        p = counts_ref[...] / _NTOK
        ent = jnp.sum(p * jnp.log(p + 1e-10))
        ppl_ref[0, 0] = jnp.exp(-ent)


def kernel(inputs, embedding_weight):
    x = jnp.transpose(inputs, (0, 2, 3, 4, 1)).reshape(_NTOK, _D)
    nsteps = _NTOK // _NB
    q, enc, idx, loss, ppl = pl.pallas_call(
        _vq_step,
        grid=(nsteps,),
        in_specs=[
            pl.BlockSpec((_NB, _D), lambda g: (g, 0)),
            pl.BlockSpec((_K, _D), lambda g: (0, 0)),
        ],
        out_specs=[
            pl.BlockSpec((_NB, _D), lambda g: (g, 0)),
            pl.BlockSpec((_NB, _K), lambda g: (g, 0)),
            pl.BlockSpec((1, 1, _NB), lambda g: (g, 0, 0)),
            pl.BlockSpec(memory_space=pltpu.SMEM),
            pl.BlockSpec(memory_space=pltpu.SMEM),
        ],
        out_shape=[
            jax.ShapeDtypeStruct((_NTOK, _D), jnp.float32),
            jax.ShapeDtypeStruct((_NTOK, _K), jnp.float32),
            jax.ShapeDtypeStruct((nsteps, 1, _NB), jnp.int32),
            jax.ShapeDtypeStruct((1, 1), jnp.float32),
            jax.ShapeDtypeStruct((1, 1), jnp.float32),
        ],
        scratch_shapes=[
            pltpu.VMEM((1, _K), jnp.float32),
            pltpu.SMEM((1,), jnp.float32),
            pltpu.VMEM((_K, 1), jnp.float32),
        ],
    )(x, embedding_weight)
    q_out = q.reshape(4, 8, 16, 16, _D).transpose(0, 4, 1, 2, 3)
    return (loss[0, 0], q_out, ppl[0, 0], enc, idx.reshape(_NTOK, 1))


# speedup vs baseline: 1.0473x; 1.0022x over previous
import jax
import jax.numpy as jnp
from jax.experimental import pallas as pl
from jax.experimental.pallas import tpu as pltpu

_K = 1024
_D = 256
_NTOK = 8192
_CC = 0.25
_NB = 2048


def _vq_step(x_ref, w_ref, q_ref, enc_ref, idx_ref, loss_ref, ppl_ref,
             counts_ref, sse_ref, wnorm_ref):
    g = pl.program_id(0)
    nsteps = pl.num_programs(0)
    w = w_ref[...]

    @pl.when(g == 0)
    def _init():
        counts_ref[...] = jnp.zeros_like(counts_ref)
        sse_ref[0] = 0.0
        wnorm_ref[...] = jnp.sum(w * w, axis=1, keepdims=True)

    x = x_ref[...]
    wnorm = wnorm_ref[...]
    xnorm = jnp.sum(x * x, axis=1, keepdims=True)
    s_t = jax.lax.dot_general(w, x, (((1,), (1,)), ((), ())),
                              preferred_element_type=jnp.float32,
                              precision=jax.lax.Precision.DEFAULT)
    dist_t = (jnp.transpose(xnorm) + wnorm) - 2.0 * s_t
    idx = jnp.argmin(dist_t, axis=0)

    iota_k = jax.lax.broadcasted_iota(jnp.int32, (_NB, _K), 1)
    idx_col = jnp.transpose(idx[None, :])
    enc = (idx_col == iota_k).astype(jnp.float32)
    q = jax.lax.dot_general(enc, w, (((1,), (0,)), ((), ())),
                            preferred_element_type=jnp.float32,
                            precision=jax.lax.Precision.DEFAULT)
    q_ref[...] = x + (q - x)
    enc_ref[...] = enc
    idx_ref[0, 0] = idx

    counts_ref[...] += jnp.sum(enc, axis=0, keepdims=True)
    sse_ref[0] += jnp.sum((q - x) ** 2)

    @pl.when(g == nsteps - 1)
    def _fini():
        loss_ref[0, 0] = _CC * sse_ref[0] / (_NTOK * _D)
        p = counts_ref[...] / _NTOK
        ent = jnp.sum(p * jnp.log(p + 1e-10))
        ppl_ref[0, 0] = jnp.exp(-ent)


def kernel(inputs, embedding_weight):
    x = jnp.transpose(inputs, (0, 2, 3, 4, 1)).reshape(_NTOK, _D)
    nsteps = _NTOK // _NB
    q, enc, idx, loss, ppl = pl.pallas_call(
        _vq_step,
        grid=(nsteps,),
        in_specs=[
            pl.BlockSpec((_NB, _D), lambda g: (g, 0)),
            pl.BlockSpec((_K, _D), lambda g: (0, 0)),
        ],
        out_specs=[
            pl.BlockSpec((_NB, _D), lambda g: (g, 0)),
            pl.BlockSpec((_NB, _K), lambda g: (g, 0)),
            pl.BlockSpec((1, 1, _NB), lambda g: (g, 0, 0)),
            pl.BlockSpec(memory_space=pltpu.SMEM),
            pl.BlockSpec(memory_space=pltpu.SMEM),
        ],
        out_shape=[
            jax.ShapeDtypeStruct((_NTOK, _D), jnp.float32),
            jax.ShapeDtypeStruct((_NTOK, _K), jnp.float32),
            jax.ShapeDtypeStruct((nsteps, 1, _NB), jnp.int32),
            jax.ShapeDtypeStruct((1, 1), jnp.float32),
            jax.ShapeDtypeStruct((1, 1), jnp.float32),
        ],
        scratch_shapes=[
            pltpu.VMEM((1, _K), jnp.float32),
            pltpu.SMEM((1,), jnp.float32),
            pltpu.VMEM((_K, 1), jnp.float32),
        ],
        compiler_params=pltpu.CompilerParams(
            vmem_limit_bytes=128 * 1024 * 1024),
    )(x, embedding_weight)
    q_out = q.reshape(4, 8, 16, 16, _D).transpose(0, 4, 1, 2, 3)
    return (loss[0, 0], q_out, ppl[0, 0], enc, idx.reshape(_NTOK, 1))
